# baseline (speedup 1.0000x reference)
import functools

import jax
import jax.numpy as jnp
from jax import lax
from jax.experimental import pallas as pl
from jax.experimental.pallas import tpu as pltpu
from jax.experimental.pallas import tpu_sc as plsc

BATCH = 16384
HIDDEN = 4096
NUM_LAYERS = 32
TOPK = 4
BLOCK_R = 1024
NUM_CORES = 2
NUM_SUBCORES = 16
NW = NUM_CORES * NUM_SUBCORES
ROWS_W = BATCH // NW
LANES = 16
GROUPS = ROWS_W // LANES


def _tc_scores_kernel(h_ref, w_ref, b_ref, out_ref):
    logits_t = jax.lax.dot_general(
        w_ref[...], h_ref[...],
        dimension_numbers=(((1,), (1,)), ((), ())),
        preferred_element_type=jnp.float32,
    ) + b_ref[...]
    sig_t = jax.nn.sigmoid(logits_t)
    out_ref[0] = sig_t[:, :ROWS_W]
    out_ref[1] = sig_t[:, ROWS_W:]


def _tc_scores(h_pooled, W, b_bcast):
    grid = (BATCH // BLOCK_R,)
    return pl.pallas_call(
        _tc_scores_kernel,
        grid=grid,
        in_specs=[
            pl.BlockSpec((BLOCK_R, HIDDEN), lambda i: (i, 0)),
            pl.BlockSpec((NUM_LAYERS, HIDDEN), lambda i: (0, 0)),
            pl.BlockSpec((NUM_LAYERS, BLOCK_R), lambda i: (0, 0)),
        ],
        out_specs=pl.BlockSpec((2, NUM_LAYERS, ROWS_W), lambda i: (i, 0, 0)),
        out_shape=jax.ShapeDtypeStruct((NW, NUM_LAYERS, ROWS_W), jnp.float32),
    )(h_pooled, W, b_bcast)


def _sc_gates_body(sig_hbm, hard_hbm, out_hbm, sig_v, gates_v, hard_v):
    wid = lax.axis_index("s") * NUM_CORES + lax.axis_index("c")
    pltpu.sync_copy(sig_hbm.at[wid], sig_v)
    pltpu.sync_copy(hard_hbm, hard_v)
    hard_mask = hard_v[...] != 0
    iota16 = lax.iota(jnp.int32, 16)
    ones = jnp.full((LANES,), 1.0, jnp.float32)
    zeros = jnp.zeros((LANES,), jnp.float32)
    i32_one = jnp.full((LANES,), 1, jnp.int32)
    i32_zero = jnp.zeros((LANES,), jnp.int32)
    neg_inf = jnp.full((LANES,), -jnp.inf, jnp.float32)

    def group(g, carry):
        vs = [sig_v[e, pl.ds(g * LANES, LANES)] for e in range(NUM_LAYERS)]
        t1 = t2 = t3 = t4 = neg_inf
        for v in vs:
            l1 = jnp.minimum(t1, v)
            t1 = jnp.maximum(t1, v)
            l2 = jnp.minimum(t2, l1)
            t2 = jnp.maximum(t2, l1)
            l3 = jnp.minimum(t3, l2)
            t3 = jnp.maximum(t3, l2)
            t4 = jnp.maximum(t4, l3)
        n_gt = i32_zero
        for v in vs:
            n_gt = n_gt + jnp.where(v > t4, i32_one, i32_zero)
        need = jnp.full((LANES,), TOPK, jnp.int32) - n_gt
        cnt = i32_zero
        row_idx = iota16 + g * LANES
        for e, v in enumerate(vs):
            take_eq = (v == t4) & (cnt < need)
            sel = (v > t4) | take_eq
            cnt = cnt + jnp.where(take_eq, i32_one, i32_zero)
            gate = jnp.where(sel, ones, zeros)
            outv = jnp.where(hard_mask, gate, v)
            plsc.store_scatter(gates_v,
                               [row_idx, jnp.full((LANES,), e, jnp.int32)],
                               outv)
        return carry

    lax.fori_loop(0, GROUPS, group, 0)
    pltpu.sync_copy(gates_v, out_hbm.at[pl.ds(wid * ROWS_W, ROWS_W), :])


_sc_gates = functools.partial(
    pl.kernel,
    out_type=jax.ShapeDtypeStruct((BATCH, NUM_LAYERS), jnp.float32),
    mesh=plsc.VectorSubcoreMesh(core_axis_name="c", subcore_axis_name="s",
                                num_cores=NUM_CORES,
                                num_subcores=NUM_SUBCORES),
    scratch_types=[
        pltpu.VMEM((NUM_LAYERS, ROWS_W), jnp.float32),
        pltpu.VMEM((ROWS_W, NUM_LAYERS), jnp.float32),
        pltpu.VMEM((LANES,), jnp.int32),
    ],
    compiler_params=pltpu.CompilerParams(needs_layout_passes=False),
)(_sc_gates_body)


def kernel(h_pooled, W, b, hard):
    b_bcast = jnp.broadcast_to(b[:, None], (NUM_LAYERS, BLOCK_R))
    hard_vec = jnp.broadcast_to(jnp.asarray(hard, jnp.int32), (LANES,))
    sig_t = _tc_scores(h_pooled, W, b_bcast)
    return _sc_gates(sig_t, hard_vec)


# speedup vs baseline: 1.0266x; 1.0266x over previous
import functools

import jax
import jax.numpy as jnp
from jax import lax
from jax.experimental import pallas as pl
from jax.experimental.pallas import tpu as pltpu
from jax.experimental.pallas import tpu_sc as plsc

BATCH = 16384
HIDDEN = 4096
NUM_LAYERS = 32
TOPK = 4
BLOCK_R = 1024
NUM_CORES = 2
NUM_SUBCORES = 16
NW = NUM_CORES * NUM_SUBCORES
ROWS_W = BATCH // NW
LANES = 16
GROUPS = ROWS_W // LANES


def _tc_scores_kernel(h_ref, w_ref, b_ref, out_ref):
    logits_t = jax.lax.dot_general(
        w_ref[...], h_ref[...],
        dimension_numbers=(((1,), (1,)), ((), ())),
        preferred_element_type=jnp.float32,
    ) + b_ref[...]
    sig_t = jax.nn.sigmoid(logits_t)
    out_ref[0] = sig_t[:, :ROWS_W]
    out_ref[1] = sig_t[:, ROWS_W:]


def _tc_scores(h_pooled, W, b_bcast):
    grid = (BATCH // BLOCK_R,)
    return pl.pallas_call(
        _tc_scores_kernel,
        grid=grid,
        in_specs=[
            pl.BlockSpec((BLOCK_R, HIDDEN), lambda i: (i, 0)),
            pl.BlockSpec((NUM_LAYERS, HIDDEN), lambda i: (0, 0)),
            pl.BlockSpec((NUM_LAYERS, BLOCK_R), lambda i: (0, 0)),
        ],
        out_specs=pl.BlockSpec((2, NUM_LAYERS, ROWS_W), lambda i: (i, 0, 0)),
        out_shape=jax.ShapeDtypeStruct((NW, NUM_LAYERS, ROWS_W), jnp.float32),
    )(h_pooled, W, b_bcast)


def _sc_gates_body(sig_hbm, hard_hbm, out_hbm, sig_v, gates_v, hard_v):
    wid = lax.axis_index("s") * NUM_CORES + lax.axis_index("c")
    pltpu.sync_copy(sig_hbm.at[wid], sig_v)
    pltpu.sync_copy(hard_hbm, hard_v)
    hard_mask = hard_v[...] != 0
    iota16 = lax.iota(jnp.int32, 16)
    ones = jnp.full((LANES,), 1.0, jnp.float32)
    zeros = jnp.zeros((LANES,), jnp.float32)
    i32_one = jnp.full((LANES,), 1, jnp.int32)
    i32_zero = jnp.zeros((LANES,), jnp.int32)
    neg_inf = jnp.full((LANES,), -jnp.inf, jnp.float32)

    @plsc.parallel_loop(0, ROWS_W, step=LANES, unroll=2)
    def group(r0):
        vs = [sig_v[e, pl.ds(r0, LANES)] for e in range(NUM_LAYERS)]
        t1 = t2 = t3 = t4 = neg_inf
        for v in vs:
            l1 = jnp.minimum(t1, v)
            t1 = jnp.maximum(t1, v)
            l2 = jnp.minimum(t2, l1)
            t2 = jnp.maximum(t2, l1)
            l3 = jnp.minimum(t3, l2)
            t3 = jnp.maximum(t3, l2)
            t4 = jnp.maximum(t4, l3)
        n_gt = (jnp.where(t1 > t4, i32_one, i32_zero)
                + jnp.where(t2 > t4, i32_one, i32_zero)
                + jnp.where(t3 > t4, i32_one, i32_zero))
        need = jnp.full((LANES,), TOPK, jnp.int32) - n_gt
        cnt = i32_zero
        row_idx = iota16 + r0
        for e, v in enumerate(vs):
            take_eq = (v == t4) & (cnt < need)
            sel = (v > t4) | take_eq
            cnt = cnt + jnp.where(take_eq, i32_one, i32_zero)
            gate = jnp.where(sel, ones, zeros)
            outv = jnp.where(hard_mask, gate, v)
            plsc.store_scatter(gates_v,
                               [row_idx, jnp.full((LANES,), e, jnp.int32)],
                               outv)
    pltpu.sync_copy(gates_v, out_hbm.at[pl.ds(wid * ROWS_W, ROWS_W), :])


_sc_gates = functools.partial(
    pl.kernel,
    out_type=jax.ShapeDtypeStruct((BATCH, NUM_LAYERS), jnp.float32),
    mesh=plsc.VectorSubcoreMesh(core_axis_name="c", subcore_axis_name="s",
                                num_cores=NUM_CORES,
                                num_subcores=NUM_SUBCORES),
    scratch_types=[
        pltpu.VMEM((NUM_LAYERS, ROWS_W), jnp.float32),
        pltpu.VMEM((ROWS_W, NUM_LAYERS), jnp.float32),
        pltpu.VMEM((LANES,), jnp.int32),
    ],
    compiler_params=pltpu.CompilerParams(needs_layout_passes=False),
)(_sc_gates_body)


def kernel(h_pooled, W, b, hard):
    b_bcast = jnp.broadcast_to(b[:, None], (NUM_LAYERS, BLOCK_R))
    hard_vec = jnp.broadcast_to(jnp.asarray(hard, jnp.int32), (LANES,))
    sig_t = _tc_scores(h_pooled, W, b_bcast)
    return _sc_gates(sig_t, hard_vec)


# speedup vs baseline: 1.0409x; 1.0140x over previous
import functools

import jax
import jax.numpy as jnp
from jax import lax
from jax.experimental import pallas as pl
from jax.experimental.pallas import tpu as pltpu
from jax.experimental.pallas import tpu_sc as plsc

BATCH = 16384
HIDDEN = 4096
NUM_LAYERS = 32
TOPK = 4
BLOCK_R = 1024
NUM_CORES = 2
NUM_SUBCORES = 16
NW = NUM_CORES * NUM_SUBCORES
LANES = 16
CHUNKS = 2
CHUNK_ROWS = BATCH // CHUNKS
ROWS_W = CHUNK_ROWS // NW
SLABS_PER_BLOCK = BLOCK_R // ROWS_W


def _tc_scores_kernel(h_ref, w_ref, b_ref, out_ref):
    logits_t = jax.lax.dot_general(
        w_ref[...], h_ref[...],
        dimension_numbers=(((1,), (1,)), ((), ())),
        preferred_element_type=jnp.float32,
    ) + b_ref[...]
    sig_t = jax.nn.sigmoid(logits_t)
    for k in range(SLABS_PER_BLOCK):
        out_ref[k] = sig_t[:, k * ROWS_W:(k + 1) * ROWS_W]


def _tc_scores(h_pooled, W, b_bcast, chunk):
    grid = (CHUNK_ROWS // BLOCK_R,)
    base = chunk * (CHUNK_ROWS // BLOCK_R)
    return pl.pallas_call(
        _tc_scores_kernel,
        grid=grid,
        in_specs=[
            pl.BlockSpec((BLOCK_R, HIDDEN), lambda i: (base + i, 0)),
            pl.BlockSpec((NUM_LAYERS, HIDDEN), lambda i: (0, 0)),
            pl.BlockSpec((NUM_LAYERS, BLOCK_R), lambda i: (0, 0)),
        ],
        out_specs=pl.BlockSpec((SLABS_PER_BLOCK, NUM_LAYERS, ROWS_W),
                               lambda i: (i, 0, 0)),
        out_shape=jax.ShapeDtypeStruct((NW, NUM_LAYERS, ROWS_W), jnp.float32),
    )(h_pooled, W, b_bcast)


def _sc_gates_body(sig_hbm, hard_hbm, out_hbm, sig_v, gates_v, hard_v):
    wid = lax.axis_index("s") * NUM_CORES + lax.axis_index("c")
    pltpu.sync_copy(sig_hbm.at[wid], sig_v)
    pltpu.sync_copy(hard_hbm, hard_v)
    hard_mask = hard_v[...] != 0
    iota16 = lax.iota(jnp.int32, 16)
    ones = jnp.full((LANES,), 1.0, jnp.float32)
    zeros = jnp.zeros((LANES,), jnp.float32)
    i32_one = jnp.full((LANES,), 1, jnp.int32)
    i32_zero = jnp.zeros((LANES,), jnp.int32)
    neg_inf = jnp.full((LANES,), -jnp.inf, jnp.float32)

    @plsc.parallel_loop(0, ROWS_W, step=LANES, unroll=2)
    def group(r0):
        vs = [sig_v[e, pl.ds(r0, LANES)] for e in range(NUM_LAYERS)]
        t1 = t2 = t3 = t4 = neg_inf
        for v in vs:
            l1 = jnp.minimum(t1, v)
            t1 = jnp.maximum(t1, v)
            l2 = jnp.minimum(t2, l1)
            t2 = jnp.maximum(t2, l1)
            l3 = jnp.minimum(t3, l2)
            t3 = jnp.maximum(t3, l2)
            t4 = jnp.maximum(t4, l3)
        n_gt = (jnp.where(t1 > t4, i32_one, i32_zero)
                + jnp.where(t2 > t4, i32_one, i32_zero)
                + jnp.where(t3 > t4, i32_one, i32_zero))
        need = jnp.full((LANES,), TOPK, jnp.int32) - n_gt
        cnt = i32_zero
        row_idx = iota16 + r0
        for e, v in enumerate(vs):
            take_eq = (v == t4) & (cnt < need)
            sel = (v > t4) | take_eq
            cnt = cnt + jnp.where(take_eq, i32_one, i32_zero)
            gate = jnp.where(sel, ones, zeros)
            outv = jnp.where(hard_mask, gate, v)
            plsc.store_scatter(gates_v,
                               [row_idx, jnp.full((LANES,), e, jnp.int32)],
                               outv)

    pltpu.sync_copy(gates_v, out_hbm.at[pl.ds(wid * ROWS_W, ROWS_W), :])


_sc_gates = functools.partial(
    pl.kernel,
    out_type=jax.ShapeDtypeStruct((CHUNK_ROWS, NUM_LAYERS), jnp.float32),
    mesh=plsc.VectorSubcoreMesh(core_axis_name="c", subcore_axis_name="s",
                                num_cores=NUM_CORES,
                                num_subcores=NUM_SUBCORES),
    scratch_types=[
        pltpu.VMEM((NUM_LAYERS, ROWS_W), jnp.float32),
        pltpu.VMEM((ROWS_W, NUM_LAYERS), jnp.float32),
        pltpu.VMEM((LANES,), jnp.int32),
    ],
    compiler_params=pltpu.CompilerParams(needs_layout_passes=False),
)(_sc_gates_body)


def kernel(h_pooled, W, b, hard):
    b_bcast = jnp.broadcast_to(b[:, None], (NUM_LAYERS, BLOCK_R))
    hard_vec = jnp.broadcast_to(jnp.asarray(hard, jnp.int32), (LANES,))
    gates = []
    for c in range(CHUNKS):
        sig_t = _tc_scores(h_pooled, W, b_bcast, c)
        gates.append(_sc_gates(sig_t, hard_vec))
    return jnp.concatenate(gates, axis=0)
